# baseline (speedup 1.0000x reference)
import functools

import jax
import jax.numpy as jnp
from jax import lax
from jax.experimental import pallas as pl
from jax.experimental.pallas import tpu as pltpu
from jax.experimental.pallas import tpu_sc as plsc

_D = 64
_B = 4096 * 200
_NC = 2
_NS = 16
_NW = _NC * _NS
_BPW = _B // _NW
_CHUNK = 128
_NCHUNK = _BPW // _CHUNK

_mesh = plsc.VectorSubcoreMesh(core_axis_name="c", subcore_axis_name="s")


@functools.partial(
    pl.kernel,
    mesh=_mesh,
    out_type=jax.ShapeDtypeStruct((_B, _D), jnp.float32),
    scratch_types=[
        pltpu.VMEM((_CHUNK,), jnp.int32),
        pltpu.VMEM((_CHUNK, _D), jnp.float32),
        pltpu.SemaphoreType.DMA,
    ],
    compiler_params=pltpu.CompilerParams(use_tc_tiling_on_sc=False),
)
def _gather(idx_hbm, table_hbm, out_hbm, idx_v, rows_v, sem):
    wid = lax.axis_index("s") * _NC + lax.axis_index("c")
    base = wid * _BPW

    def body(i, carry):
        off = base + i * _CHUNK
        pltpu.sync_copy(idx_hbm.at[pl.ds(off, _CHUNK)], idx_v)
        pltpu.async_copy(table_hbm.at[idx_v], rows_v, sem).wait()
        pltpu.sync_copy(rows_v, out_hbm.at[pl.ds(off, _CHUNK)])
        return carry

    lax.fori_loop(0, _NCHUNK, body, 0)


def kernel(traj, table):
    idx = traj.reshape(-1).astype(jnp.int32)
    out = _gather(idx, table)
    return out.reshape(traj.shape[0], traj.shape[1], _D)


# speedup vs baseline: 1.1950x; 1.1950x over previous
import functools

import jax
import jax.numpy as jnp
from jax import lax
from jax.experimental import pallas as pl
from jax.experimental.pallas import tpu as pltpu
from jax.experimental.pallas import tpu_sc as plsc

_D = 64
_B = 4096 * 200
_NC = 2
_NS = 16
_NW = _NC * _NS
_BPW = _B // _NW
_CHUNK = 128
_K = 4
_GROUP = _CHUNK * _K
_NGROUP = _BPW // _GROUP
_HALF = _NGROUP // 2

_mesh = plsc.VectorSubcoreMesh(core_axis_name="c", subcore_axis_name="s")


@functools.partial(
    pl.kernel,
    mesh=_mesh,
    out_type=jax.ShapeDtypeStruct((_B, _D), jnp.float32),
    scratch_types=[
        pltpu.VMEM((_BPW,), jnp.int32),
        pltpu.VMEM((_GROUP, _D), jnp.float32),
        pltpu.VMEM((_GROUP, _D), jnp.float32),
        pltpu.SemaphoreType.DMA,
        pltpu.SemaphoreType.DMA,
        pltpu.SemaphoreType.DMA,
        pltpu.SemaphoreType.DMA,
    ],
    compiler_params=pltpu.CompilerParams(use_tc_tiling_on_sc=False),
)
def _gather(idx_hbm, table_hbm, out_hbm, idx_v, rows0, rows1, g0, g1, s0, s1):
    wid = lax.axis_index("s") * _NC + lax.axis_index("c")
    base = wid * _BPW

    pltpu.sync_copy(idx_hbm.at[pl.ds(base, _BPW)], idx_v)

    def fire_group(t, rows, gsem):
        for j in range(_K):
            pltpu.async_copy(
                table_hbm.at[idx_v.at[pl.ds(t * _GROUP + j * _CHUNK, _CHUNK)]],
                rows.at[pl.ds(j * _CHUNK, _CHUNK)],
                gsem,
            )

    def drain_group(rows, gsem):
        pltpu.make_async_copy(table_hbm.at[idx_v.at[pl.ds(0, _CHUNK)]],
                              rows.at[pl.ds(0, _CHUNK)], gsem).wait()
        pltpu.make_async_copy(table_hbm.at[idx_v.at[pl.ds(0, _CHUNK)]],
                              rows.at[pl.ds(_CHUNK, _CHUNK)], gsem).wait()
        pltpu.make_async_copy(table_hbm.at[idx_v.at[pl.ds(0, _CHUNK)]],
                              rows.at[pl.ds(2 * _CHUNK, _CHUNK)], gsem).wait()
        pltpu.make_async_copy(table_hbm.at[idx_v.at[pl.ds(0, _CHUNK)]],
                              rows.at[pl.ds(3 * _CHUNK, _CHUNK)], gsem).wait()

    def fire_store(t, rows, ssem):
        pltpu.async_copy(rows, out_hbm.at[pl.ds(base + t * _GROUP, _GROUP)], ssem)

    def wait_store(t, rows, ssem):
        pltpu.make_async_copy(rows, out_hbm.at[pl.ds(base + t * _GROUP, _GROUP)],
                              ssem).wait()

    fire_group(0, rows0, g0)

    def body(tt, carry):
        t = tt * 2
        drain_group(rows0, g0)

        @pl.when(t > 0)
        def _():
            wait_store(t - 1, rows1, s1)

        fire_group(t + 1, rows1, g1)
        fire_store(t, rows0, s0)

        drain_group(rows1, g1)
        wait_store(t, rows0, s0)

        @pl.when(t + 2 < _NGROUP)
        def _():
            fire_group(t + 2, rows0, g0)

        fire_store(t + 1, rows1, s1)
        return carry

    lax.fori_loop(0, _HALF, body, 0)
    wait_store(_NGROUP - 1, rows1, s1)


def kernel(traj, table):
    idx = traj.reshape(-1).astype(jnp.int32)
    out = _gather(idx, table)
    return out.reshape(traj.shape[0], traj.shape[1], _D)
